# baseline (speedup 1.0000x reference)
import jax
import jax.numpy as jnp
from jax import lax
from jax.experimental import pallas as pl
from jax.experimental.pallas import tpu as pltpu
from jax.experimental.pallas import tpu_sc as plsc

IMG = 14
P = IMG * IMG
D = 512
B = 8
TAU = 0.5 / P
BETA = 4.0
NS = 16
L = 16
CPW = D // NS
NV = CPW // L


def _sc_body(x_hbm, out_hbm, xv, outv, zv, sem_in, sem_z, sem_out):
    wid = lax.axis_index("s")
    base = wid * CPW

    cp_in = pltpu.async_copy(x_hbm.at[:, pl.ds(base, CPW)], xv, sem_in)

    def zstep(p, _):
        for k in range(NV):
            zv[p, pl.ds(k * L, L)] = jnp.zeros((L,), jnp.float32)
        return 0

    lax.fori_loop(0, P, zstep, 0, unroll=IMG)

    zcps = [
        pltpu.async_copy(zv, out_hbm.at[b, :, pl.ds(base, CPW)], sem_z)
        for b in range(1, B)
    ]

    cp_in.wait()

    def amax_step(p, carry):
        maxs, idxs = carry
        idx = jnp.full((L,), p, jnp.int32)
        new_maxs, new_idxs = [], []
        for k in range(NV):
            v = xv[p, pl.ds(k * L, L)]
            pred = v > maxs[k]
            new_maxs.append(jnp.where(pred, v, maxs[k]))
            new_idxs.append(jnp.where(pred, idx, idxs[k]))
        return (tuple(new_maxs), tuple(new_idxs))

    init = (
        tuple(jnp.full((L,), -jnp.inf, jnp.float32) for _ in range(NV)),
        tuple(jnp.zeros((L,), jnp.int32) for _ in range(NV)),
    )
    _, mu = lax.fori_loop(0, P, amax_step, init, unroll=IMG)
    img_v = jnp.full((L,), IMG, jnp.int32)
    rows = [lax.div(mu[k], img_v).astype(jnp.float32) for k in range(NV)]
    cols = [lax.rem(mu[k], img_v).astype(jnp.float32) for k in range(NV)]

    def mask_row(i, _):
        i_f = jnp.full((L,), i, jnp.int32).astype(jnp.float32)
        dis = [jnp.abs(i_f - rows[k]) for k in range(NV)]

        def mask_col(j, _):
            j_f = jnp.full((L,), j, jnp.int32).astype(jnp.float32)
            p = i * IMG + j
            for k in range(NV):
                dist = dis[k] + jnp.abs(j_f - cols[k])
                m = TAU * jnp.maximum(1.0 - (BETA / IMG) * dist, -1.0)
                outv[p, pl.ds(k * L, L)] = xv[p, pl.ds(k * L, L)] * m
            return 0

        lax.fori_loop(0, IMG, mask_col, 0, unroll=IMG)
        return 0

    lax.fori_loop(0, IMG, mask_row, 0)

    cp_out = pltpu.async_copy(outv, out_hbm.at[0, :, pl.ds(base, CPW)], sem_out)
    for cp in zcps:
        cp.wait()
    cp_out.wait()


_sc_call = pl.kernel(
    _sc_body,
    out_type=jax.ShapeDtypeStruct((B, P, D), jnp.float32),
    mesh=plsc.VectorSubcoreMesh(
        core_axis_name="c", subcore_axis_name="s", num_cores=1
    ),
    scratch_types=[
        pltpu.VMEM((P, CPW), jnp.float32),
        pltpu.VMEM((P, CPW), jnp.float32),
        pltpu.VMEM((P, CPW), jnp.float32),
        pltpu.SemaphoreType.DMA,
        pltpu.SemaphoreType.DMA,
        pltpu.SemaphoreType.DMA,
    ],
    compiler_params=pltpu.CompilerParams(use_tc_tiling_on_sc=False),
)


@jax.jit
def kernel(x):
    x0 = x[0].reshape(P, D)
    out = _sc_call(x0)
    return out.reshape(B, IMG, IMG, D)


# speedup vs baseline: 1.1252x; 1.1252x over previous
import jax
import jax.numpy as jnp
from jax import lax
from jax.experimental import pallas as pl
from jax.experimental.pallas import tpu as pltpu
from jax.experimental.pallas import tpu_sc as plsc

IMG = 14
P = IMG * IMG
D = 512
B = 8
TAU = 0.5 / P
BETA = 4.0
NS = 16
L = 16
CPW = D // NS
NV = CPW // L


def _sc_body(x_hbm, out_hbm, xv, outv, sem_in, sem_out):
    wid = lax.axis_index("s")
    base = wid * CPW

    pltpu.async_copy(x_hbm.at[:, pl.ds(base, CPW)], xv, sem_in).wait()

    def amax_step(p, carry):
        maxs, idxs = carry
        idx = jnp.full((L,), p, jnp.int32)
        new_maxs, new_idxs = [], []
        for k in range(NV):
            v = xv[p, pl.ds(k * L, L)]
            pred = v > maxs[k]
            new_maxs.append(jnp.where(pred, v, maxs[k]))
            new_idxs.append(jnp.where(pred, idx, idxs[k]))
        return (tuple(new_maxs), tuple(new_idxs))

    init = (
        tuple(jnp.full((L,), -jnp.inf, jnp.float32) for _ in range(NV)),
        tuple(jnp.zeros((L,), jnp.int32) for _ in range(NV)),
    )
    _, mu = lax.fori_loop(0, P, amax_step, init, unroll=IMG)
    img_v = jnp.full((L,), IMG, jnp.int32)
    rows = [lax.div(mu[k], img_v).astype(jnp.float32) for k in range(NV)]
    cols = [lax.rem(mu[k], img_v).astype(jnp.float32) for k in range(NV)]

    def mask_row(i, _):
        i_f = jnp.full((L,), i, jnp.int32).astype(jnp.float32)
        dis = [jnp.abs(i_f - rows[k]) for k in range(NV)]

        def mask_col(j, _):
            j_f = jnp.full((L,), j, jnp.int32).astype(jnp.float32)
            p = i * IMG + j
            for k in range(NV):
                dist = dis[k] + jnp.abs(j_f - cols[k])
                m = TAU * jnp.maximum(1.0 - (BETA / IMG) * dist, -1.0)
                outv[p, pl.ds(k * L, L)] = xv[p, pl.ds(k * L, L)] * m
            return 0

        lax.fori_loop(0, IMG, mask_col, 0, unroll=IMG)
        return 0

    lax.fori_loop(0, IMG, mask_row, 0)

    pltpu.async_copy(outv, out_hbm.at[:, pl.ds(base, CPW)], sem_out).wait()


_sc_call = pl.kernel(
    _sc_body,
    out_type=jax.ShapeDtypeStruct((P, D), jnp.float32),
    mesh=plsc.VectorSubcoreMesh(
        core_axis_name="c", subcore_axis_name="s", num_cores=1
    ),
    scratch_types=[
        pltpu.VMEM((P, CPW), jnp.float32),
        pltpu.VMEM((P, CPW), jnp.float32),
        pltpu.SemaphoreType.DMA,
        pltpu.SemaphoreType.DMA,
    ],
    compiler_params=pltpu.CompilerParams(use_tc_tiling_on_sc=False),
)


@jax.jit
def kernel(x):
    x0 = x[0].reshape(P, D)
    out0 = _sc_call(x0)
    out = jnp.zeros((B, P, D), dtype=x.dtype).at[0].set(out0)
    return out.reshape(B, IMG, IMG, D)


# speedup vs baseline: 1.1489x; 1.0210x over previous
import jax
import jax.numpy as jnp
from jax import lax
from jax.experimental import pallas as pl
from jax.experimental.pallas import tpu as pltpu
from jax.experimental.pallas import tpu_sc as plsc

IMG = 14
P = IMG * IMG
D = 512
B = 8
TAU = 0.5 / P
BETA = 4.0
NC = 2
NS = 16
L = 16
NW = NC * NS
CPW = D // NW
NV = CPW // L


def _sc_body(x_hbm, out_hbm, xv, outv, sem_in, sem_out):
    wid = lax.axis_index("s") * NC + lax.axis_index("c")
    base = wid * CPW

    pltpu.async_copy(x_hbm.at[:, pl.ds(base, CPW)], xv, sem_in).wait()

    def amax_step(p, carry):
        maxs, idxs = carry
        idx = jnp.full((L,), p, jnp.int32)
        new_maxs, new_idxs = [], []
        for k in range(NV):
            v = xv[p, pl.ds(k * L, L)]
            pred = v > maxs[k]
            new_maxs.append(jnp.where(pred, v, maxs[k]))
            new_idxs.append(jnp.where(pred, idx, idxs[k]))
        return (tuple(new_maxs), tuple(new_idxs))

    init = (
        tuple(jnp.full((L,), -jnp.inf, jnp.float32) for _ in range(NV)),
        tuple(jnp.zeros((L,), jnp.int32) for _ in range(NV)),
    )
    _, mu = lax.fori_loop(0, P, amax_step, init, unroll=IMG)
    img_v = jnp.full((L,), IMG, jnp.int32)
    rows = [lax.div(mu[k], img_v).astype(jnp.float32) for k in range(NV)]
    cols = [lax.rem(mu[k], img_v).astype(jnp.float32) for k in range(NV)]

    def mask_row(i, _):
        i_f = jnp.full((L,), i, jnp.int32).astype(jnp.float32)
        dis = [jnp.abs(i_f - rows[k]) for k in range(NV)]

        def mask_col(j, _):
            j_f = jnp.full((L,), j, jnp.int32).astype(jnp.float32)
            p = i * IMG + j
            for k in range(NV):
                dist = dis[k] + jnp.abs(j_f - cols[k])
                m = TAU * jnp.maximum(1.0 - (BETA / IMG) * dist, -1.0)
                outv[p, pl.ds(k * L, L)] = xv[p, pl.ds(k * L, L)] * m
            return 0

        lax.fori_loop(0, IMG, mask_col, 0, unroll=IMG)
        return 0

    lax.fori_loop(0, IMG, mask_row, 0)

    pltpu.async_copy(outv, out_hbm.at[:, pl.ds(base, CPW)], sem_out).wait()


_sc_call = pl.kernel(
    _sc_body,
    out_type=jax.ShapeDtypeStruct((P, D), jnp.float32),
    mesh=plsc.VectorSubcoreMesh(core_axis_name="c", subcore_axis_name="s"),
    scratch_types=[
        pltpu.VMEM((P, CPW), jnp.float32),
        pltpu.VMEM((P, CPW), jnp.float32),
        pltpu.SemaphoreType.DMA,
        pltpu.SemaphoreType.DMA,
    ],
    compiler_params=pltpu.CompilerParams(use_tc_tiling_on_sc=False),
)


@jax.jit
def kernel(x):
    x0 = x[0].reshape(P, D)
    out0 = _sc_call(x0)
    out = jnp.zeros((B, P, D), dtype=x.dtype).at[0].set(out0)
    return out.reshape(B, IMG, IMG, D)
